# baseline (speedup 1.0000x reference)
import functools

import jax
import jax.numpy as jnp
from jax import lax
from jax.experimental import pallas as pl
from jax.experimental.pallas import tpu as pltpu
from jax.experimental.pallas import tpu_sc as plsc

_H_UPPER = 10.0
_B = 128
_N = 32768
_BN = 4096
_NB = _N // _BN


def _main_body(h_ref, hm_ref, hv_ref,
               out_ref, mnew_ref, cidx_ref, fidx_ref,
               mnew_s, vnew_s, colneg_s, svar, rmax, ridx):
    p = pl.program_id(0)
    j = pl.program_id(1)
    ds = pl.ds(j * _BN, _BN)

    @pl.when(p == 0)
    def _phase0():
        xb = h_ref[...]
        out_ref[...] = xb
        ones = jnp.full((1, _B), 1.0 / _B, jnp.float32)
        mu = jnp.dot(ones, xb, preferred_element_type=jnp.float32)
        msq = jnp.dot(ones, xb * xb, preferred_element_type=jnp.float32)
        var = msq - mu * mu
        hm = hm_ref[...]
        hv = hv_ref[...]
        mn = (hm * _H_UPPER + mu) / (_H_UPPER + 1.0)
        vn = (hv * (_H_UPPER - 1.0 / _B) + var
              + (mu - hm) ** 2 / (1.0 + 1.0 / _H_UPPER)) \
            / (_H_UPPER + 1.0 - 1.0 / _B)
        mnew_s[:, ds] = mn
        vnew_s[:, ds] = vn
        mnew_ref[...] = mn

        @pl.when(j == 0)
        def _():
            svar[0, 0] = 0.0
            colneg_s[...] = -lax.broadcasted_iota(
                jnp.int32, (1, _BN), 1).astype(jnp.float32)
        svar[0, 0] += jnp.sum(vn)

    @pl.when(p == 1)
    def _phase1():
        xb = h_ref[...]
        mb = mnew_s[:, ds]
        vb = vnew_s[:, ds]
        c = svar[0, 0] / (float(_N) * 100.0)
        rinv = 1.0 / (vb + c)
        d = xb - mb
        score = d * d * rinv
        bmax = jnp.max(score, axis=1, keepdims=True)
        cn = colneg_s[...] - (j * _BN).astype(jnp.float32)
        cand = jnp.where(score == bmax, cn, -jnp.inf)
        barg = jnp.max(cand, axis=1, keepdims=True)

        @pl.when(j == 0)
        def _():
            rmax[...] = bmax
            ridx[...] = barg

        @pl.when(j != 0)
        def _():
            better = bmax > rmax[...]
            rmax[...] = jnp.where(better, bmax, rmax[...])
            ridx[...] = jnp.where(better, barg, ridx[...])

        @pl.when(j == _NB - 1)
        def _():
            coli = (-ridx[...]).astype(jnp.int32)
            cidx_ref[...] = coli
            fidx_ref[...] = coli + lax.broadcasted_iota(
                jnp.int32, (_B, 1), 0) * _N


def _build_main(interpret):
    return pl.pallas_call(
        _main_body,
        grid=(2, _NB),
        in_specs=[
            pl.BlockSpec((_B, _BN), lambda p, j: (0, j)),
            pl.BlockSpec((1, _BN), lambda p, j: (0, jnp.where(p == 0, j, _NB - 1))),
            pl.BlockSpec((1, _BN), lambda p, j: (0, jnp.where(p == 0, j, _NB - 1))),
        ],
        out_specs=[
            pl.BlockSpec((_B, _BN), lambda p, j: (0, jnp.where(p == 0, j, _NB - 1))),
            pl.BlockSpec((1, _BN), lambda p, j: (0, jnp.where(p == 0, j, _NB - 1))),
            pl.BlockSpec((_B, 1), lambda p, j: (0, 0)),
            pl.BlockSpec((_B, 1), lambda p, j: (0, 0)),
        ],
        out_shape=[
            jax.ShapeDtypeStruct((_B, _N), jnp.float32),
            jax.ShapeDtypeStruct((1, _N), jnp.float32),
            jax.ShapeDtypeStruct((_B, 1), jnp.int32),
            jax.ShapeDtypeStruct((_B, 1), jnp.int32),
        ],
        scratch_shapes=[
            pltpu.VMEM((1, _N), jnp.float32),
            pltpu.VMEM((1, _N), jnp.float32),
            pltpu.VMEM((1, _BN), jnp.float32),
            pltpu.SMEM((1, 1), jnp.float32),
            pltpu.VMEM((_B, 1), jnp.float32),
            pltpu.VMEM((_B, 1), jnp.float32),
        ],
        compiler_params=pltpu.CompilerParams(
            dimension_semantics=("arbitrary", "arbitrary"),
        ),
        interpret=interpret,
    )


_NWORK = _B // 16


@functools.lru_cache(maxsize=None)
def _get_sc_fixup():
    mesh = plsc.VectorSubcoreMesh(core_axis_name="c", subcore_axis_name="s")

    @functools.partial(
        pl.kernel,
        out_type=(),
        mesh=mesh,
        scratch_types=[
            pltpu.VMEM((16,), jnp.int32),
            pltpu.VMEM((16,), jnp.int32),
            pltpu.VMEM((16,), jnp.float32),
            pltpu.SemaphoreType.DMA,
            pltpu.SemaphoreType.DMA,
        ],
    )
    def _sc_fixup(out_flat, mnew_flat, cidx_flat, fidx_flat,
                  civ, fiv, valv, sem1, sem2):
        cid = lax.axis_index("c")
        sid = lax.axis_index("s")
        w = sid * 2 + cid

        @pl.when(w < _NWORK)
        def _():
            base = w * 16
            pltpu.sync_copy(cidx_flat.at[pl.ds(base, 16)], civ)
            pltpu.sync_copy(fidx_flat.at[pl.ds(base, 16)], fiv)
            pltpu.async_copy(mnew_flat.at[civ], valv, sem1).wait()
            pltpu.async_copy(valv, out_flat.at[fiv], sem2).wait()

    return _sc_fixup


def kernel(h, h_mean, h_var):
    out, mnew, cidx, fidx = _build_main(False)(h, h_mean, h_var)
    out_ref = jax.new_ref(out.reshape(_B * _N))
    _get_sc_fixup()(out_ref, mnew.reshape(_N), cidx.reshape(_B), fidx.reshape(_B))
    return out_ref[...].reshape(_B, _N)


# speedup vs baseline: 2.5062x; 2.5062x over previous
import jax
import jax.numpy as jnp
from jax import lax
from jax.experimental import pallas as pl
from jax.experimental.pallas import tpu as pltpu

_H_UPPER = 10.0
_B = 128
_N = 32768
_BN = 4096
_NB = _N // _BN


def _body(h_ref, hm_ref, hv_ref, out_ref,
          mnew_s, vnew_s, colneg_s, svar, rmax, ridx,
          obuf, cvm, csm, pbuf, mbuf, osem0, osem1, dsem, psem, msem):
    p = pl.program_id(0)
    j = pl.program_id(1)
    ds = pl.ds(j * _BN, _BN)

    def _oc(slot, dst_ds, sem):
        return pltpu.make_async_copy(obuf.at[slot], out_ref.at[:, dst_ds], sem)

    @pl.when(p == 0)
    def _phase0():
        xb = h_ref[...]
        slot = lax.rem(j, 2)

        @pl.when(j >= 2)
        def _():
            @pl.when(slot == 0)
            def _():
                _oc(0, pl.ds(0, _BN), osem0).wait()

            @pl.when(slot == 1)
            def _():
                _oc(1, pl.ds(0, _BN), osem1).wait()

        obuf[slot] = xb

        @pl.when(slot == 0)
        def _():
            _oc(0, ds, osem0).start()

        @pl.when(slot == 1)
        def _():
            _oc(1, ds, osem1).start()

        ones = jnp.full((1, _B), 1.0 / _B, jnp.float32)
        mu = jnp.dot(ones, xb, preferred_element_type=jnp.float32)
        msq = jnp.dot(ones, xb * xb, preferred_element_type=jnp.float32)
        var = msq - mu * mu
        hm = hm_ref[...]
        hv = hv_ref[...]
        mn = (hm * _H_UPPER + mu) / (_H_UPPER + 1.0)
        vn = (hv * (_H_UPPER - 1.0 / _B) + var
              + (mu - hm) ** 2 / (1.0 + 1.0 / _H_UPPER)) \
            / (_H_UPPER + 1.0 - 1.0 / _B)
        mnew_s[:, ds] = mn
        vnew_s[:, ds] = vn

        @pl.when(j == 0)
        def _():
            svar[0, 0] = 0.0
            colneg_s[...] = -lax.broadcasted_iota(
                jnp.int32, (1, _BN), 1).astype(jnp.float32)
        svar[0, 0] += jnp.sum(vn)

    @pl.when(p == 1)
    def _phase1():
        xb = h_ref[...]
        mb = mnew_s[:, ds]
        vb = vnew_s[:, ds]
        c = svar[0, 0] / (float(_N) * 100.0)
        rinv = 1.0 / (vb + c)
        d = xb - mb
        score = d * d * rinv
        bmax = jnp.max(score, axis=1, keepdims=True)
        cn = colneg_s[...] - (j * _BN).astype(jnp.float32)
        cand = jnp.where(score == bmax, cn, -jnp.inf)
        barg = jnp.max(cand, axis=1, keepdims=True)

        @pl.when(j == 0)
        def _():
            rmax[...] = bmax
            ridx[...] = barg

        @pl.when(j != 0)
        def _():
            better = bmax > rmax[...]
            rmax[...] = jnp.where(better, bmax, rmax[...])
            ridx[...] = jnp.where(better, barg, ridx[...])

        @pl.when(j == _NB - 1)
        def _patch():
            cvm[...] = (-ridx[...]).astype(jnp.int32)
            pltpu.make_async_copy(cvm, csm, dsem).start()
            _oc(0, pl.ds(0, _BN), osem0).wait()
            _oc(1, pl.ds(0, _BN), osem1).wait()
            pltpu.make_async_copy(cvm, csm, dsem).wait()

            def _fire(b, carry):
                col = csm[b, 0]
                c8 = (col // 128) * 128
                pltpu.make_async_copy(
                    out_ref.at[pl.ds(b, 1), pl.ds(c8, 128)],
                    pbuf.at[pl.ds(b, 1), :], psem).start()
                pltpu.make_async_copy(
                    mnew_s.at[:, pl.ds(c8, 128)],
                    mbuf.at[pl.ds(b, 1), :], msem).start()
                return carry

            lax.fori_loop(0, _B, _fire, 0)

            def _drain1(b, carry):
                pltpu.make_async_copy(
                    out_ref.at[pl.ds(0, 1), pl.ds(0, 128)],
                    pbuf.at[pl.ds(0, 1), :], psem).wait()
                pltpu.make_async_copy(
                    mnew_s.at[:, pl.ds(0, 128)],
                    mbuf.at[pl.ds(0, 1), :], msem).wait()
                return carry

            lax.fori_loop(0, _B, _drain1, 0)

            lane = lax.broadcasted_iota(jnp.int32, (_B, 128), 1)
            cm = cvm[...] & 127
            pbuf[...] = jnp.where(lane == cm, mbuf[...], pbuf[...])

            def _fire2(b, carry):
                col = csm[b, 0]
                c8 = (col // 128) * 128
                pltpu.make_async_copy(
                    pbuf.at[pl.ds(b, 1), :],
                    out_ref.at[pl.ds(b, 1), pl.ds(c8, 128)], psem).start()
                return carry

            lax.fori_loop(0, _B, _fire2, 0)

            def _drain2(b, carry):
                pltpu.make_async_copy(
                    pbuf.at[pl.ds(0, 1), :],
                    out_ref.at[pl.ds(0, 1), pl.ds(0, 128)], psem).wait()
                return carry

            lax.fori_loop(0, _B, _drain2, 0)


def _build(interpret):
    return pl.pallas_call(
        _body,
        grid=(2, _NB),
        in_specs=[
            pl.BlockSpec((_B, _BN), lambda p, j: (0, j)),
            pl.BlockSpec((1, _BN), lambda p, j: (0, jnp.where(p == 0, j, _NB - 1))),
            pl.BlockSpec((1, _BN), lambda p, j: (0, jnp.where(p == 0, j, _NB - 1))),
        ],
        out_specs=pl.BlockSpec(memory_space=pltpu.MemorySpace.HBM),
        out_shape=jax.ShapeDtypeStruct((_B, _N), jnp.float32),
        scratch_shapes=[
            pltpu.VMEM((1, _N), jnp.float32),
            pltpu.VMEM((1, _N), jnp.float32),
            pltpu.VMEM((1, _BN), jnp.float32),
            pltpu.SMEM((1, 1), jnp.float32),
            pltpu.VMEM((_B, 1), jnp.float32),
            pltpu.VMEM((_B, 1), jnp.float32),
            pltpu.VMEM((2, _B, _BN), jnp.float32),
            pltpu.VMEM((_B, 1), jnp.int32),
            pltpu.SMEM((_B, 1), jnp.int32),
            pltpu.VMEM((_B, 128), jnp.float32),
            pltpu.VMEM((_B, 128), jnp.float32),
            pltpu.SemaphoreType.DMA,
            pltpu.SemaphoreType.DMA,
            pltpu.SemaphoreType.DMA,
            pltpu.SemaphoreType.DMA,
            pltpu.SemaphoreType.DMA,
        ],
        compiler_params=pltpu.CompilerParams(
            dimension_semantics=("arbitrary", "arbitrary"),
        ),
        interpret=interpret,
    )


@jax.jit
def kernel(h, h_mean, h_var):
    return _build(False)(h, h_mean, h_var)


# speedup vs baseline: 3.3071x; 1.3196x over previous
import jax
import jax.numpy as jnp
from jax import lax
from jax.experimental import pallas as pl
from jax.experimental.pallas import tpu as pltpu

_H_UPPER = 10.0
_B = 128
_N = 32768
_BN = 4096
_NB = _N // _BN


def _body(h_ref, hm_ref, hv_ref, out_ref,
          hbuf, mnew_s, vnew_s, colneg_s, svar, rmax, ridx):
    p = pl.program_id(0)
    j = pl.program_id(1)
    ds = pl.ds(j * _BN, _BN)

    @pl.when(p == 0)
    def _phase0():
        xb = h_ref[...]
        hbuf[:, ds] = xb
        ones = jnp.full((1, _B), 1.0 / _B, jnp.float32)
        mu = jnp.dot(ones, xb, preferred_element_type=jnp.float32)
        msq = jnp.dot(ones, xb * xb, preferred_element_type=jnp.float32)
        var = msq - mu * mu
        hm = hm_ref[...]
        hv = hv_ref[...]
        mn = (hm * _H_UPPER + mu) / (_H_UPPER + 1.0)
        vn = (hv * (_H_UPPER - 1.0 / _B) + var
              + (mu - hm) ** 2 / (1.0 + 1.0 / _H_UPPER)) \
            / (_H_UPPER + 1.0 - 1.0 / _B)
        mnew_s[:, ds] = mn
        vnew_s[:, ds] = vn

        @pl.when(j == 0)
        def _():
            svar[0, 0] = 0.0
            colneg_s[...] = -lax.broadcasted_iota(
                jnp.int32, (1, _BN), 1).astype(jnp.float32)
        svar[0, 0] += jnp.sum(vn)

    @pl.when(p == 1)
    def _phase1():
        xb = hbuf[:, ds]
        mb = mnew_s[:, ds]
        vb = vnew_s[:, ds]
        c = svar[0, 0] / (float(_N) * 100.0)
        rinv = 1.0 / (vb + c)
        d = xb - mb
        score = d * d * rinv
        bmax = jnp.max(score, axis=1, keepdims=True)
        cn = colneg_s[...] - (j * _BN).astype(jnp.float32)
        cand = jnp.where(score == bmax, cn, -jnp.inf)
        barg = jnp.max(cand, axis=1, keepdims=True)

        @pl.when(j == 0)
        def _():
            rmax[...] = bmax
            ridx[...] = barg

        @pl.when(j != 0)
        def _():
            better = bmax > rmax[...]
            rmax[...] = jnp.where(better, bmax, rmax[...])
            ridx[...] = jnp.where(better, barg, ridx[...])

    @pl.when(p == 2)
    def _phase2():
        xb = hbuf[:, ds]
        mb = mnew_s[:, ds]
        cn = colneg_s[...] - (j * _BN).astype(jnp.float32)
        sel = cn == ridx[...]
        out_ref[...] = jnp.where(sel, jnp.broadcast_to(mb, xb.shape), xb)


def _build(interpret):
    return pl.pallas_call(
        _body,
        grid=(3, _NB),
        in_specs=[
            pl.BlockSpec((_B, _BN), lambda p, j: (0, jnp.where(p == 0, j, 0))),
            pl.BlockSpec((1, _BN), lambda p, j: (0, jnp.where(p == 0, j, 0))),
            pl.BlockSpec((1, _BN), lambda p, j: (0, jnp.where(p == 0, j, 0))),
        ],
        out_specs=pl.BlockSpec((_B, _BN), lambda p, j: (0, jnp.where(p == 2, j, 0))),
        out_shape=jax.ShapeDtypeStruct((_B, _N), jnp.float32),
        scratch_shapes=[
            pltpu.VMEM((_B, _N), jnp.float32),
            pltpu.VMEM((1, _N), jnp.float32),
            pltpu.VMEM((1, _N), jnp.float32),
            pltpu.VMEM((1, _BN), jnp.float32),
            pltpu.SMEM((1, 1), jnp.float32),
            pltpu.VMEM((_B, 1), jnp.float32),
            pltpu.VMEM((_B, 1), jnp.float32),
        ],
        compiler_params=pltpu.CompilerParams(
            dimension_semantics=("arbitrary", "arbitrary"),
        ),
        interpret=interpret,
    )


@jax.jit
def kernel(h, h_mean, h_var):
    return _build(False)(h, h_mean, h_var)


# speedup vs baseline: 3.8493x; 1.1640x over previous
import jax
import jax.numpy as jnp
from jax import lax
from jax.experimental import pallas as pl
from jax.experimental.pallas import tpu as pltpu

_H_UPPER = 10.0
_B = 128
_N = 32768
_BN = 8192
_NB = _N // _BN


def _body(h_ref, hm_ref, hv_ref, out_ref,
          hbuf, mnew_s, vnew_s, colneg_s, svar, rmax, ridx):
    p = pl.program_id(0)
    j = pl.program_id(1)
    ds = pl.ds(j * _BN, _BN)

    @pl.when(p == 0)
    def _phase0():
        xb = h_ref[...]
        hbuf[:, ds] = xb
        ones = jnp.full((1, _B), 1.0 / _B, jnp.float32)
        mu = jnp.dot(ones, xb, preferred_element_type=jnp.float32)
        msq = jnp.dot(ones, xb * xb, preferred_element_type=jnp.float32)
        var = msq - mu * mu
        hm = hm_ref[...]
        hv = hv_ref[...]
        mn = (hm * _H_UPPER + mu) / (_H_UPPER + 1.0)
        vn = (hv * (_H_UPPER - 1.0 / _B) + var
              + (mu - hm) ** 2 / (1.0 + 1.0 / _H_UPPER)) \
            / (_H_UPPER + 1.0 - 1.0 / _B)
        mnew_s[:, ds] = mn
        vnew_s[:, ds] = vn

        @pl.when(j == 0)
        def _():
            svar[0, 0] = 0.0
            colneg_s[...] = -lax.broadcasted_iota(
                jnp.int32, (1, _BN), 1).astype(jnp.float32)
        svar[0, 0] += jnp.sum(vn)

    @pl.when(p == 1)
    def _phase1():
        xb = hbuf[:, ds]
        mb = mnew_s[:, ds]
        vb = vnew_s[:, ds]
        c = svar[0, 0] / (float(_N) * 100.0)
        rinv = 1.0 / (vb + c)
        d = xb - mb
        score = d * d * rinv
        bmax = jnp.max(score, axis=1, keepdims=True)
        cn = colneg_s[...] - (j * _BN).astype(jnp.float32)
        cand = jnp.where(score == bmax, cn, -jnp.inf)
        barg = jnp.max(cand, axis=1, keepdims=True)

        @pl.when(j == 0)
        def _():
            rmax[...] = bmax
            ridx[...] = barg

        @pl.when(j != 0)
        def _():
            better = bmax > rmax[...]
            rmax[...] = jnp.where(better, bmax, rmax[...])
            ridx[...] = jnp.where(better, barg, ridx[...])

    @pl.when(p == 2)
    def _phase2():
        xb = hbuf[:, ds]
        mb = mnew_s[:, ds]
        cn = colneg_s[...] - (j * _BN).astype(jnp.float32)
        sel = cn == ridx[...]
        out_ref[...] = jnp.where(sel, jnp.broadcast_to(mb, xb.shape), xb)


def _build(interpret):
    return pl.pallas_call(
        _body,
        grid=(3, _NB),
        in_specs=[
            pl.BlockSpec((_B, _BN), lambda p, j: (0, jnp.where(p == 0, j, 0))),
            pl.BlockSpec((1, _BN), lambda p, j: (0, jnp.where(p == 0, j, 0))),
            pl.BlockSpec((1, _BN), lambda p, j: (0, jnp.where(p == 0, j, 0))),
        ],
        out_specs=pl.BlockSpec((_B, _BN), lambda p, j: (0, jnp.where(p == 2, j, 0))),
        out_shape=jax.ShapeDtypeStruct((_B, _N), jnp.float32),
        scratch_shapes=[
            pltpu.VMEM((_B, _N), jnp.float32),
            pltpu.VMEM((1, _N), jnp.float32),
            pltpu.VMEM((1, _N), jnp.float32),
            pltpu.VMEM((1, _BN), jnp.float32),
            pltpu.SMEM((1, 1), jnp.float32),
            pltpu.VMEM((_B, 1), jnp.float32),
            pltpu.VMEM((_B, 1), jnp.float32),
        ],
        compiler_params=pltpu.CompilerParams(
            dimension_semantics=("arbitrary", "arbitrary"),
        ),
        interpret=interpret,
    )


@jax.jit
def kernel(h, h_mean, h_var):
    return _build(False)(h, h_mean, h_var)
